# baseline (speedup 1.0000x reference)
import functools

import jax
import jax.numpy as jnp
from jax import lax
from jax.experimental import pallas as pl
from jax.experimental.pallas import tpu as pltpu
from jax.experimental.pallas import tpu_sc as plsc

_BATCH = 16384
_DIMS = (32, 16, 16, 16, 16)
_COLS = (0, 32, 48, 64, 80)
_OUT_D = 96
_CHUNK = 128
_NC, _NS = 2, 16
_NW = _NC * _NS
_BPW = _BATCH // _NW
_NCH = _BPW // _CHUNK


def _build():
    mesh = plsc.VectorSubcoreMesh(core_axis_name="c", subcore_axis_name="s")

    scratch = []
    for d in _DIMS:
        scratch.append(pltpu.VMEM((_NCH, _CHUNK), jnp.int32))
        scratch.append(pltpu.VMEM((_BPW, d), jnp.float32))
        scratch.append(pltpu.SemaphoreType.DMA)
    scratch.append(pltpu.SemaphoreType.DMA)

    @functools.partial(
        pl.kernel,
        out_type=jax.ShapeDtypeStruct((_BATCH, _OUT_D), jnp.float32),
        mesh=mesh,
        scratch_types=scratch,
        compiler_params=pltpu.CompilerParams(use_tc_tiling_on_sc=False),
    )
    def user_feat_sc(i0, i1, i2, i3, i4, w0, w1, w2, w3, w4, out,
                     x0, r0, s0, x1, r1, s1, x2, r2, s2,
                     x3, r3, s3, x4, r4, s4, sw):
        wid = lax.axis_index("s") * _NC + lax.axis_index("c")
        base = wid * _BPW
        idx_hbm = (i0, i1, i2, i3, i4)
        tables = (w0, w1, w2, w3, w4)
        idx_v = (x0, x1, x2, x3, x4)
        rows_v = (r0, r1, r2, r3, r4)
        sems = (s0, s1, s2, s3, s4)

        idx_cps = []
        for t in range(5):
            cp = pltpu.make_async_copy(
                idx_hbm[t].at[pl.ds(wid * _NCH, _NCH)], idx_v[t], sems[t])
            cp.start()
            idx_cps.append(cp)

        gather_cps = []
        for t in range(5):
            idx_cps[t].wait()
            for j in range(_NCH):
                cp = pltpu.make_async_copy(
                    tables[t].at[idx_v[t].at[j]],
                    rows_v[t].at[pl.ds(j * _CHUNK, _CHUNK)],
                    sems[t])
                cp.start()
                gather_cps.append(cp)

        write_cps = []
        for t in range(5):
            for j in range(_NCH):
                gather_cps[t * _NCH + j].wait()
            cp = pltpu.make_async_copy(
                rows_v[t],
                out.at[pl.ds(base, _BPW), pl.ds(_COLS[t], _DIMS[t])],
                sw)
            cp.start()
            write_cps.append(cp)
        for cp in write_cps:
            cp.wait()

    return user_feat_sc


_FN = None


def kernel(idx_id, idx_gender, idx_age, idx_occupation, idx_city,
           W_id, W_gender, W_age, W_occupation, W_city):
    global _FN
    if _FN is None:
        _FN = _build()
    idxs = [x.reshape(_NW * _NCH, _CHUNK)
            for x in (idx_id, idx_gender, idx_age, idx_occupation, idx_city)]
    return _FN(*idxs, W_id, W_gender, W_age, W_occupation, W_city)
